# baseline (speedup 1.0000x reference)
import functools

import jax
import jax.numpy as jnp
from jax.experimental import pallas as pl
from jax.experimental.pallas import tpu as pltpu

_NUM_PATCH = 6
_NUM_BAND = 5
_NUM_NODES = 12
_SUMMARY = 11
_D = 128
_H = 8
_DH = 16

_NBRS = (
    [[(s, 0) for s in range(6, 11)] + [(_SUMMARY, 2)] for _ in range(6)]
    + [[(s, 0) for s in range(0, 6)] + [(_SUMMARY, 2)] for _ in range(5)]
    + [[(s, 1) for s in range(0, 11)]]
)
_NBRS = {d: nb for d, nb in enumerate(_NBRS)}

_BLOCK_KEYS = ("ln1_g", "ln1_b", "WqkvT", "bqkv",
               "WoT", "bo", "tbiasT", "ln2_g", "ln2_b", "W1T", "b1", "W2T", "b2")

_BF = jnp.bfloat16
_F32 = jnp.float32


def _mm(w, x):
    return jnp.dot(w, x, preferred_element_type=jnp.float32)


def _layer_norm_cols(x, g, b):
    mu = jnp.mean(x, axis=0, keepdims=True)
    xc = x - mu
    var = jnp.mean(xc * xc, axis=0, keepdims=True)
    return xc * jax.lax.rsqrt(var + 1e-5) * g + b


def _body(nblocks, Bt, *refs):
    it = iter(refs)
    x_in = next(it)
    sum_tok = next(it)
    blocks = []
    for _ in range(nblocks):
        blocks.append({k: next(it) for k in _BLOCK_KEYS})
    wpT = next(it)
    bp = next(it)
    wbT = next(it)
    bb = next(it)
    projWT = next(it)
    projb = next(it)
    plng = next(it)
    plnb = next(it)
    out_ref = next(it)
    prod_ref = next(it)
    alpha_ref = next(it)

    hrow = jax.lax.broadcasted_iota(jnp.int32, (_H, _D), 0)
    hcol = jax.lax.broadcasted_iota(jnp.int32, (_H, _D), 1)
    Hpool = (hcol // _DH == hrow).astype(_BF)
    Hexp = jnp.transpose(Hpool)

    slabs = [x_in[n] for n in range(11)]
    slabs.append(jnp.broadcast_to(sum_tok[...], (_D, Bt)))
    x = jnp.concatenate(slabs, axis=1)

    pairs = []
    for d in range(_NUM_NODES):
        for (s, et) in _NBRS[d]:
            pairs.append((d, s, et))
    nE = len(pairs)

    for p in blocks:
        h = _layer_norm_cols(x, p["ln1_g"][...], p["ln1_b"][...]).astype(_BF)
        qkv = (_mm(p["WqkvT"][...], h) + p["bqkv"][...]).astype(_BF)
        qs = [qkv[0:_D, n * Bt:(n + 1) * Bt] for n in range(_NUM_NODES)]
        ks = [qkv[_D:2 * _D, n * Bt:(n + 1) * Bt] for n in range(_NUM_NODES)]
        vs = [qkv[2 * _D:3 * _D, n * Bt:(n + 1) * Bt] for n in range(_NUM_NODES)]
        tb = p["tbiasT"][...]

        for g, (d, s, et) in enumerate(pairs):
            prod_ref[:, g * Bt:(g + 1) * Bt] = qs[d] * ks[s]
        logits = _mm(Hpool, prod_ref[...])
        g0 = 0
        for d in range(_NUM_NODES):
            nbrs = _NBRS[d]
            es = [jnp.exp(logits[:, (g0 + j) * Bt:(g0 + j + 1) * Bt] + tb[:, et:et + 1])
                  for j, (s, et) in enumerate(nbrs)]
            den = es[0]
            for e in es[1:]:
                den = den + e
            inv = 1.0 / (den + 1e-9)
            for j, e in enumerate(es):
                alpha_ref[:, (g0 + j) * Bt:(g0 + j + 1) * Bt] = (e * inv).astype(_BF)
            g0 += len(nbrs)
        groups = [(0, 36), (36, 71), (71, 82)]
        aexp = [None] * nE
        for (a, b) in groups:
            ex = _mm(Hexp, alpha_ref[:, a * Bt:b * Bt]).astype(_BF)
            for g in range(a, b):
                aexp[g] = ex[:, (g - a) * Bt:(g - a + 1) * Bt]
        aggs = []
        g0 = 0
        for d in range(_NUM_NODES):
            nbrs = _NBRS[d]
            agg = None
            for j, (s, et) in enumerate(nbrs):
                t = aexp[g0 + j] * vs[s]
                agg = t if agg is None else agg + t
            aggs.append(agg)
            g0 += len(nbrs)
        agg_all = jnp.concatenate(aggs, axis=1)

        x = x + _mm(p["WoT"][...], agg_all) + p["bo"][...]
        h2 = _layer_norm_cols(x, p["ln2_g"][...], p["ln2_b"][...]).astype(_BF)
        a1 = jax.nn.gelu(_mm(p["W1T"][...], h2).astype(_BF) + p["b1"][...])
        x = x + _mm(p["W2T"][...], a1) + p["b2"][...]

    summary_out = x[:, _SUMMARY * Bt:(_SUMMARY + 1) * Bt]
    gate_p = jax.nn.sigmoid(
        _mm(wpT[...], x[:, :_NUM_PATCH * Bt]) + bp[...])
    gate_b = jax.nn.sigmoid(
        _mm(wbT[...], x[:, _NUM_PATCH * Bt:_SUMMARY * Bt]) + bb[...])
    pool_p = None
    for n in range(_NUM_PATCH):
        t = x[:, n * Bt:(n + 1) * Bt] * gate_p[:, n * Bt:(n + 1) * Bt]
        pool_p = t if pool_p is None else pool_p + t
    pool_b = None
    for j in range(_NUM_BAND):
        n = _NUM_PATCH + j
        t = x[:, n * Bt:(n + 1) * Bt] * gate_b[:, j * Bt:(j + 1) * Bt]
        pool_b = t if pool_b is None else pool_b + t

    comb = jnp.concatenate([summary_out, pool_p, pool_b], axis=0)
    o = _mm(projWT[...], comb) + projb[...]
    o = _layer_norm_cols(o, plng[...], plnb[...])
    out_ref[...] = jax.nn.gelu(o)


def kernel(patch_tokens, band_tokens, params):
    B = patch_tokens.shape[0]
    D = patch_tokens.shape[-1]
    dh = D // _H
    Bt = 512 if B % 512 == 0 else B
    grid = B // Bt

    x0 = jnp.concatenate([jnp.transpose(patch_tokens, (1, 2, 0)),
                          jnp.transpose(band_tokens, (1, 2, 0))], axis=0)

    scale = 1.0 / (dh ** 0.5)
    bf = jnp.bfloat16
    arrays = [x0, params["summary_token"].reshape(D, 1)]
    for p in params["blocks"]:
        arrays += [
            p["ln1_g"].reshape(D, 1), p["ln1_b"].reshape(D, 1),
            jnp.concatenate([p["Wq"].T * scale, p["Wk"].T, p["Wv"].T], axis=0).astype(bf),
            jnp.concatenate([p["bq"] * scale, p["bk"], p["bv"]], axis=0).reshape(3 * D, 1),
            p["Wo"].T.astype(bf), p["bo"].reshape(D, 1),
            p["tbias"].T,
            p["ln2_g"].reshape(D, 1), p["ln2_b"].reshape(D, 1),
            p["W1"].T.astype(bf), p["b1"].reshape(4 * D, 1).astype(bf),
            p["W2"].T.astype(bf), p["b2"].reshape(D, 1),
        ]
    arrays += [
        params["patch_gate_w"].T, params["patch_gate_b"].reshape(1, 1),
        params["band_gate_w"].T, params["band_gate_b"].reshape(1, 1),
        params["proj_W"].T, params["proj_b"].reshape(D, 1),
        params["proj_ln_g"].reshape(D, 1), params["proj_ln_b"].reshape(D, 1),
    ]

    in_specs = [pl.BlockSpec((11, D, Bt), lambda i: (0, 0, i))]
    in_specs += [pl.BlockSpec(a.shape, functools.partial(lambda nd, i: (0,) * nd, a.ndim))
                 for a in arrays[1:]]

    out = pl.pallas_call(
        functools.partial(_body, len(params["blocks"]), Bt),
        grid=(grid,),
        in_specs=in_specs,
        out_specs=pl.BlockSpec((D, Bt), lambda i: (0, i)),
        out_shape=jax.ShapeDtypeStruct((D, B), jnp.float32),
        scratch_shapes=[pltpu.VMEM((D, 82 * Bt), jnp.bfloat16),
                        pltpu.VMEM((8, 82 * Bt), jnp.bfloat16)],
        compiler_params=pltpu.CompilerParams(dimension_semantics=("parallel",)),
    )(*arrays)
    return out.T


# speedup vs baseline: 1.0005x; 1.0005x over previous
import functools

import jax
import jax.numpy as jnp
from jax.experimental import pallas as pl
from jax.experimental.pallas import tpu as pltpu

_NUM_PATCH = 6
_NUM_BAND = 5
_NUM_NODES = 12
_SUMMARY = 11
_D = 128
_H = 8
_DH = 16

_NBRS = (
    [[(s, 0) for s in range(6, 11)] + [(_SUMMARY, 2)] for _ in range(6)]
    + [[(s, 0) for s in range(0, 6)] + [(_SUMMARY, 2)] for _ in range(5)]
    + [[(s, 1) for s in range(0, 11)]]
)
_NBRS = {d: nb for d, nb in enumerate(_NBRS)}

_BLOCK_KEYS = ("ln1_g", "ln1_b", "WqkvT", "bqkv",
               "WoT", "bo", "tbiasT", "ln2_g", "ln2_b", "W1T", "b1", "W2T", "b2")

_BF = jnp.bfloat16
_F32 = jnp.float32


def _mm(w, x):
    return jnp.dot(w, x, preferred_element_type=jnp.float32)


def _layer_norm_cols(x, g, b):
    mu = jnp.mean(x, axis=0, keepdims=True)
    xc = x - mu
    var = jnp.mean(xc * xc, axis=0, keepdims=True)
    return xc * jax.lax.rsqrt(var + 1e-5) * g + b


def _body(nblocks, Bt, *refs):
    it = iter(refs)
    x_in = next(it)
    sum_tok = next(it)
    blocks = []
    for _ in range(nblocks):
        blocks.append({k: next(it) for k in _BLOCK_KEYS})
    wpT = next(it)
    bp = next(it)
    wbT = next(it)
    bb = next(it)
    projWT = next(it)
    projb = next(it)
    plng = next(it)
    plnb = next(it)
    out_ref = next(it)
    prod_ref = next(it)
    alpha_ref = next(it)

    pr = jax.lax.broadcasted_iota(jnp.int32, (2 * _H, 2 * _D), 0)
    pc = jax.lax.broadcasted_iota(jnp.int32, (2 * _H, 2 * _D), 1)
    Hpool = ((pc // _D == pr // _H) & ((pc % _D) // _DH == pr % _H)).astype(_BF)
    er = jax.lax.broadcasted_iota(jnp.int32, (2 * _D, 2 * _H), 0)
    ec = jax.lax.broadcasted_iota(jnp.int32, (2 * _D, 2 * _H), 1)
    Hexp = ((er // _D == ec // _H) & ((er % _D) // _DH == ec % _H)).astype(_BF)

    slabs = [x_in[n] for n in range(11)]
    slabs.append(jnp.broadcast_to(sum_tok[...], (_D, Bt)))
    x = jnp.concatenate(slabs, axis=1)

    pairs = []
    for d in range(_NUM_NODES):
        for (s, et) in _NBRS[d]:
            pairs.append((d, s, et))
    nE = len(pairs)

    for p in blocks:
        h = _layer_norm_cols(x, p["ln1_g"][...], p["ln1_b"][...]).astype(_BF)
        qkv = (_mm(p["WqkvT"][...], h) + p["bqkv"][...]).astype(_BF)
        qs = [qkv[0:_D, n * Bt:(n + 1) * Bt] for n in range(_NUM_NODES)]
        ks = [qkv[_D:2 * _D, n * Bt:(n + 1) * Bt] for n in range(_NUM_NODES)]
        vs = [qkv[2 * _D:3 * _D, n * Bt:(n + 1) * Bt] for n in range(_NUM_NODES)]
        tb = p["tbiasT"][...]

        for g, (d, s, et) in enumerate(pairs):
            r0 = _D * (g % 2)
            c0 = (g // 2) * Bt
            prod_ref[r0:r0 + _D, c0:c0 + Bt] = qs[d] * ks[s]
        logits = _mm(Hpool, prod_ref[...])

        def _plog(g):
            return logits[_H * (g % 2):_H * (g % 2) + _H,
                          (g // 2) * Bt:(g // 2 + 1) * Bt]

        g0 = 0
        for d in range(_NUM_NODES):
            nbrs = _NBRS[d]
            es = [jnp.exp(_plog(g0 + j) + tb[:, et:et + 1])
                  for j, (s, et) in enumerate(nbrs)]
            den = es[0]
            for e in es[1:]:
                den = den + e
            inv = 1.0 / (den + 1e-9)
            for j, e in enumerate(es):
                g = g0 + j
                alpha_ref[_H * (g % 2):_H * (g % 2) + _H,
                          (g // 2) * Bt:(g // 2 + 1) * Bt] = (e * inv).astype(_BF)
            g0 += len(nbrs)
        aexp = [None] * nE
        for (a, b) in [(0, 14), (14, 28), (28, 41)]:
            ex = _mm(Hexp, alpha_ref[:, a * Bt:b * Bt]).astype(_BF)
            for slot in range(a, b):
                for half in range(2):
                    g = slot * 2 + half
                    if g < nE:
                        aexp[g] = ex[_D * half:_D * half + _D,
                                     (slot - a) * Bt:(slot - a + 1) * Bt]
        aggs = []
        g0 = 0
        for d in range(_NUM_NODES):
            nbrs = _NBRS[d]
            agg = None
            for j, (s, et) in enumerate(nbrs):
                t = aexp[g0 + j] * vs[s]
                agg = t if agg is None else agg + t
            aggs.append(agg)
            g0 += len(nbrs)
        agg_all = jnp.concatenate(aggs, axis=1)

        x = x + _mm(p["WoT"][...], agg_all) + p["bo"][...]
        h2 = _layer_norm_cols(x, p["ln2_g"][...], p["ln2_b"][...]).astype(_BF)
        a1 = jax.nn.gelu(_mm(p["W1T"][...], h2).astype(_BF) + p["b1"][...])
        x = x + _mm(p["W2T"][...], a1) + p["b2"][...]

    summary_out = x[:, _SUMMARY * Bt:(_SUMMARY + 1) * Bt]
    gate_p = jax.nn.sigmoid(
        _mm(wpT[...], x[:, :_NUM_PATCH * Bt]) + bp[...])
    gate_b = jax.nn.sigmoid(
        _mm(wbT[...], x[:, _NUM_PATCH * Bt:_SUMMARY * Bt]) + bb[...])
    pool_p = None
    for n in range(_NUM_PATCH):
        t = x[:, n * Bt:(n + 1) * Bt] * gate_p[:, n * Bt:(n + 1) * Bt]
        pool_p = t if pool_p is None else pool_p + t
    pool_b = None
    for j in range(_NUM_BAND):
        n = _NUM_PATCH + j
        t = x[:, n * Bt:(n + 1) * Bt] * gate_b[:, j * Bt:(j + 1) * Bt]
        pool_b = t if pool_b is None else pool_b + t

    comb = jnp.concatenate([summary_out, pool_p, pool_b], axis=0)
    o = _mm(projWT[...], comb) + projb[...]
    o = _layer_norm_cols(o, plng[...], plnb[...])
    out_ref[...] = jax.nn.gelu(o)


def kernel(patch_tokens, band_tokens, params):
    B = patch_tokens.shape[0]
    D = patch_tokens.shape[-1]
    dh = D // _H
    Bt = 512 if B % 512 == 0 else B
    grid = B // Bt

    x0 = jnp.concatenate([jnp.transpose(patch_tokens, (1, 2, 0)),
                          jnp.transpose(band_tokens, (1, 2, 0))], axis=0)

    scale = 1.0 / (dh ** 0.5)
    bf = jnp.bfloat16
    arrays = [x0, params["summary_token"].reshape(D, 1)]
    for p in params["blocks"]:
        arrays += [
            p["ln1_g"].reshape(D, 1), p["ln1_b"].reshape(D, 1),
            jnp.concatenate([p["Wq"].T * scale, p["Wk"].T, p["Wv"].T], axis=0).astype(bf),
            jnp.concatenate([p["bq"] * scale, p["bk"], p["bv"]], axis=0).reshape(3 * D, 1),
            p["Wo"].T.astype(bf), p["bo"].reshape(D, 1),
            p["tbias"].T,
            p["ln2_g"].reshape(D, 1), p["ln2_b"].reshape(D, 1),
            p["W1"].T.astype(bf), p["b1"].reshape(4 * D, 1).astype(bf),
            p["W2"].T.astype(bf), p["b2"].reshape(D, 1),
        ]
    arrays += [
        params["patch_gate_w"].T, params["patch_gate_b"].reshape(1, 1),
        params["band_gate_w"].T, params["band_gate_b"].reshape(1, 1),
        params["proj_W"].T, params["proj_b"].reshape(D, 1),
        params["proj_ln_g"].reshape(D, 1), params["proj_ln_b"].reshape(D, 1),
    ]

    in_specs = [pl.BlockSpec((11, D, Bt), lambda i: (0, 0, i))]
    in_specs += [pl.BlockSpec(a.shape, functools.partial(lambda nd, i: (0,) * nd, a.ndim))
                 for a in arrays[1:]]

    out = pl.pallas_call(
        functools.partial(_body, len(params["blocks"]), Bt),
        grid=(grid,),
        in_specs=in_specs,
        out_specs=pl.BlockSpec((D, Bt), lambda i: (0, i)),
        out_shape=jax.ShapeDtypeStruct((D, B), jnp.float32),
        scratch_shapes=[pltpu.VMEM((2 * D, 41 * Bt), jnp.bfloat16),
                        pltpu.VMEM((16, 41 * Bt), jnp.bfloat16)],
        compiler_params=pltpu.CompilerParams(dimension_semantics=("parallel",)),
    )(*arrays)
    return out.T


# speedup vs baseline: 1.1355x; 1.1349x over previous
import functools

import jax
import jax.numpy as jnp
from jax.experimental import pallas as pl
from jax.experimental.pallas import tpu as pltpu

_NUM_PATCH = 6
_NUM_BAND = 5
_NUM_NODES = 12
_SUMMARY = 11
_D = 128
_H = 8
_DH = 16

_NBRS = (
    [[(s, 0) for s in range(6, 11)] + [(_SUMMARY, 2)] for _ in range(6)]
    + [[(s, 0) for s in range(0, 6)] + [(_SUMMARY, 2)] for _ in range(5)]
    + [[(s, 1) for s in range(0, 11)]]
)
_NBRS = {d: nb for d, nb in enumerate(_NBRS)}

_BLOCK_KEYS = ("ln1_g", "ln1_b", "WqkvT", "bqkv",
               "WoT", "bo", "tbiasT", "ln2_g", "ln2_b", "W1T", "b1", "W2T", "b2")

_BF = jnp.bfloat16
_F32 = jnp.float32


def _mm(w, x):
    return jnp.dot(w, x, preferred_element_type=jnp.float32)


def _layer_norm_cols(x, g, b):
    mu = jnp.mean(x, axis=0, keepdims=True)
    xc = x - mu
    var = jnp.mean(xc * xc, axis=0, keepdims=True)
    return xc * jax.lax.rsqrt(var + 1e-5) * g + b


def _body(nblocks, Bt, *refs):
    it = iter(refs)
    x_in = next(it)
    sum_tok = next(it)
    blocks = []
    for _ in range(nblocks):
        blocks.append({k: next(it) for k in _BLOCK_KEYS})
    wpT = next(it)
    bp = next(it)
    wbT = next(it)
    bb = next(it)
    projWT = next(it)
    projb = next(it)
    plng = next(it)
    plnb = next(it)
    out_ref = next(it)
    prod_ref = next(it)

    pr = jax.lax.broadcasted_iota(jnp.int32, (2 * _H, 2 * _D), 0)
    pc = jax.lax.broadcasted_iota(jnp.int32, (2 * _H, 2 * _D), 1)
    Hpool = ((pc // _D == pr // _H) & ((pc % _D) // _DH == pr % _H)).astype(_BF)

    slabs = [x_in[n] for n in range(11)]
    slabs.append(jnp.broadcast_to(sum_tok[...], (_D, Bt)))
    x = jnp.concatenate(slabs, axis=1)

    pairs = []
    for d in range(_NUM_NODES):
        for (s, et) in _NBRS[d]:
            pairs.append((d, s, et))
    nE = len(pairs)

    for p in blocks:
        h = _layer_norm_cols(x, p["ln1_g"][...], p["ln1_b"][...]).astype(_BF)
        qkv = (_mm(p["WqkvT"][...], h) + p["bqkv"][...]).astype(_BF)
        qs = [qkv[0:_D, n * Bt:(n + 1) * Bt] for n in range(_NUM_NODES)]
        ks = [qkv[_D:2 * _D, n * Bt:(n + 1) * Bt] for n in range(_NUM_NODES)]
        vs = [qkv[2 * _D:3 * _D, n * Bt:(n + 1) * Bt] for n in range(_NUM_NODES)]
        tb = p["tbiasT"][...]

        for g, (d, s, et) in enumerate(pairs):
            r0 = _D * (g % 2)
            c0 = (g // 2) * Bt
            prod_ref[r0:r0 + _D, c0:c0 + Bt] = qs[d] * ks[s]
        logits = _mm(Hpool, prod_ref[...])

        def _plog(g):
            return logits[_H * (g % 2):_H * (g % 2) + _H,
                          (g // 2) * Bt:(g // 2 + 1) * Bt]

        aggs = []
        g0 = 0
        for d in range(_NUM_NODES):
            nbrs = _NBRS[d]
            es = [jnp.exp(_plog(g0 + j) + tb[:, et:et + 1])
                  for j, (s, et) in enumerate(nbrs)]
            den = es[0]
            for e in es[1:]:
                den = den + e
            inv = 1.0 / (den + 1e-9)
            agg = None
            for j, (s, et) in enumerate(nbrs):
                a_bf = (es[j] * inv).astype(_BF)
                v_s = vs[s]
                t = jnp.concatenate(
                    [v_s[h * _DH:(h + 1) * _DH, :] * a_bf[h:h + 1, :]
                     for h in range(_H)], axis=0)
                agg = t if agg is None else agg + t
            aggs.append(agg)
            g0 += len(nbrs)
        agg_all = jnp.concatenate(aggs, axis=1)

        x = x + _mm(p["WoT"][...], agg_all) + p["bo"][...]
        h2 = _layer_norm_cols(x, p["ln2_g"][...], p["ln2_b"][...]).astype(_BF)
        a1 = jax.nn.gelu(_mm(p["W1T"][...], h2).astype(_BF) + p["b1"][...])
        x = x + _mm(p["W2T"][...], a1) + p["b2"][...]

    summary_out = x[:, _SUMMARY * Bt:(_SUMMARY + 1) * Bt]
    gate_p = jax.nn.sigmoid(
        _mm(wpT[...], x[:, :_NUM_PATCH * Bt]) + bp[...])
    gate_b = jax.nn.sigmoid(
        _mm(wbT[...], x[:, _NUM_PATCH * Bt:_SUMMARY * Bt]) + bb[...])
    pool_p = None
    for n in range(_NUM_PATCH):
        t = x[:, n * Bt:(n + 1) * Bt] * gate_p[:, n * Bt:(n + 1) * Bt]
        pool_p = t if pool_p is None else pool_p + t
    pool_b = None
    for j in range(_NUM_BAND):
        n = _NUM_PATCH + j
        t = x[:, n * Bt:(n + 1) * Bt] * gate_b[:, j * Bt:(j + 1) * Bt]
        pool_b = t if pool_b is None else pool_b + t

    comb = jnp.concatenate([summary_out, pool_p, pool_b], axis=0)
    o = _mm(projWT[...], comb) + projb[...]
    o = _layer_norm_cols(o, plng[...], plnb[...])
    out_ref[...] = jax.nn.gelu(o)


def kernel(patch_tokens, band_tokens, params):
    B = patch_tokens.shape[0]
    D = patch_tokens.shape[-1]
    dh = D // _H
    Bt = 512 if B % 512 == 0 else B
    grid = B // Bt

    x0 = jnp.concatenate([jnp.transpose(patch_tokens, (1, 2, 0)),
                          jnp.transpose(band_tokens, (1, 2, 0))], axis=0)

    scale = 1.0 / (dh ** 0.5)
    bf = jnp.bfloat16
    arrays = [x0, params["summary_token"].reshape(D, 1)]
    for p in params["blocks"]:
        arrays += [
            p["ln1_g"].reshape(D, 1), p["ln1_b"].reshape(D, 1),
            jnp.concatenate([p["Wq"].T * scale, p["Wk"].T, p["Wv"].T], axis=0).astype(bf),
            jnp.concatenate([p["bq"] * scale, p["bk"], p["bv"]], axis=0).reshape(3 * D, 1),
            p["Wo"].T.astype(bf), p["bo"].reshape(D, 1),
            p["tbias"].T,
            p["ln2_g"].reshape(D, 1), p["ln2_b"].reshape(D, 1),
            p["W1"].T.astype(bf), p["b1"].reshape(4 * D, 1).astype(bf),
            p["W2"].T.astype(bf), p["b2"].reshape(D, 1),
        ]
    arrays += [
        params["patch_gate_w"].T, params["patch_gate_b"].reshape(1, 1),
        params["band_gate_w"].T, params["band_gate_b"].reshape(1, 1),
        params["proj_W"].T, params["proj_b"].reshape(D, 1),
        params["proj_ln_g"].reshape(D, 1), params["proj_ln_b"].reshape(D, 1),
    ]

    in_specs = [pl.BlockSpec((11, D, Bt), lambda i: (0, 0, i))]
    in_specs += [pl.BlockSpec(a.shape, functools.partial(lambda nd, i: (0,) * nd, a.ndim))
                 for a in arrays[1:]]

    out = pl.pallas_call(
        functools.partial(_body, len(params["blocks"]), Bt),
        grid=(grid,),
        in_specs=in_specs,
        out_specs=pl.BlockSpec((D, Bt), lambda i: (0, i)),
        out_shape=jax.ShapeDtypeStruct((D, B), jnp.float32),
        scratch_shapes=[pltpu.VMEM((2 * D, 41 * Bt), jnp.bfloat16)],
        compiler_params=pltpu.CompilerParams(dimension_semantics=("parallel",)),
    )(*arrays)
    return out.T
